# baseline (speedup 1.0000x reference)
import functools

import jax
import jax.numpy as jnp
from jax import lax
from jax.experimental import pallas as pl
from jax.experimental.pallas import tpu as pltpu
from jax.experimental.pallas import tpu_sc as plsc

T, N, V = 2048, 16, 1024
BLANK = V - 1
BT = 128
GRID = T // BT
LANES = 16


def _stage1_body(lens_ref, x_ref, amax_ref, keep_ref, msum_ref, olen_ref,
                 prev_ref):
    step = pl.program_id(0)
    x = x_ref[...]
    m = jnp.max(x, axis=1)
    iota_v = lax.broadcasted_iota(jnp.int32, x.shape, 1).astype(jnp.float32)
    a_flat = jnp.min(jnp.where(x == m[:, None], iota_v, float(V)),
                     axis=1).astype(jnp.int32)
    s = jnp.sum(jnp.exp(x), axis=1)
    mlp = (m - jnp.log(s)).reshape(BT, N)
    a = a_flat.reshape(BT, N)

    carry = jnp.where(step == 0, jnp.full((1, N), -1, jnp.int32),
                      prev_ref[0:1, :N])
    a_prev = jnp.concatenate([carry, a[:-1, :]], axis=0)
    prev_ref[0:1, :N] = a[-1:, :]

    tloc = lax.broadcasted_iota(jnp.int32, (BT, N), 0) + step * BT
    tmask = tloc < lens_ref[...]
    keep = (a != BLANK) & (a != a_prev) & tmask

    amax_ref[...] = a.T
    keep_ref[...] = keep.astype(jnp.int32).T

    @pl.when(step == 0)
    def _():
        msum_ref[...] = jnp.zeros((1, N), jnp.float32)
        olen_ref[...] = jnp.zeros((1, N), jnp.int32)

    msum_ref[...] += jnp.sum(jnp.where(tmask, mlp, 0.0), axis=0)[None, :]
    olen_ref[...] += jnp.sum(keep.astype(jnp.int32), axis=0)[None, :]


def _stage1(logits, lens_row, interpret=False):
    return pl.pallas_call(
        _stage1_body,
        grid=(GRID,),
        in_specs=[
            pl.BlockSpec((1, N), lambda i: (0, 0)),
            pl.BlockSpec((BT * N, V), lambda i: (i, 0)),
        ],
        out_specs=[
            pl.BlockSpec((N, BT), lambda i: (0, i)),
            pl.BlockSpec((N, BT), lambda i: (0, i)),
            pl.BlockSpec((1, N), lambda i: (0, 0)),
            pl.BlockSpec((1, N), lambda i: (0, 0)),
        ],
        out_shape=[
            jax.ShapeDtypeStruct((N, T), jnp.int32),
            jax.ShapeDtypeStruct((N, T), jnp.int32),
            jax.ShapeDtypeStruct((1, N), jnp.float32),
            jax.ShapeDtypeStruct((1, N), jnp.int32),
        ],
        scratch_shapes=[pltpu.VMEM((8, 128), jnp.int32)],
        interpret=interpret,
    )(lens_row, logits.reshape(T * N, V))


def _stage2(amax_nt, keep_nt):
    mesh = plsc.VectorSubcoreMesh(core_axis_name="c", subcore_axis_name="s")

    @functools.partial(
        pl.kernel,
        out_type=jax.ShapeDtypeStruct((N, T), jnp.int32),
        mesh=mesh,
        scratch_types=[
            pltpu.VMEM((T,), jnp.int32),
            pltpu.VMEM((T,), jnp.int32),
        ],
        compiler_params=pltpu.CompilerParams(needs_layout_passes=False),
    )
    def sc_kernel(amax_hbm, keep_hbm, paths_hbm, a_v, k_v):
        cid = lax.axis_index("c")
        sid = lax.axis_index("s")
        row = cid * 8 + sid

        @pl.when(sid < 8)
        def _():
            pltpu.sync_copy(amax_hbm.at[row], a_v)
            pltpu.sync_copy(keep_hbm.at[row], k_v)

            def chunk(i, cnt):
                for u in range(2):
                    off = (2 * i + u) * LANES
                    a = a_v[pl.ds(off, LANES)]
                    k = k_v[pl.ds(off, LANES)]
                    c = plsc.cumsum(k)
                    pos = cnt + c - 1
                    plsc.store_scatter(a_v, [pos], a, mask=k != 0)
                    cnt = cnt + c[LANES - 1]
                return cnt

            lax.fori_loop(0, T // LANES // 2, chunk, jnp.int32(0))
            pltpu.sync_copy(a_v, paths_hbm.at[row])

    return sc_kernel(amax_nt, keep_nt)


def kernel(logits, in_lens):
    lens_row = in_lens.reshape(1, N)
    amax_nt, keep_nt, msum, olen = _stage1(logits, lens_row)
    paths_nt = _stage2(amax_nt, keep_nt)
    return (msum.reshape(N), paths_nt.T, olen.reshape(N))


# speedup vs baseline: 1.0206x; 1.0206x over previous
import functools

import jax
import jax.numpy as jnp
from jax import lax
from jax.experimental import pallas as pl
from jax.experimental.pallas import tpu as pltpu
from jax.experimental.pallas import tpu_sc as plsc

T, N, V = 2048, 16, 1024
BLANK = V - 1
BT = 128
GRID = T // BT
LANES = 16


def _stage1_body(lens_ref, x_ref, amax_ref, msum_ref, olen_ref, prev_ref):
    step = pl.program_id(0)
    x = x_ref[...]
    m = jnp.max(x, axis=1)
    iota_v = lax.broadcasted_iota(jnp.int32, x.shape, 1).astype(jnp.float32)
    a_flat = jnp.min(jnp.where(x == m[:, None], iota_v, float(V)),
                     axis=1).astype(jnp.int32)
    s = jnp.sum(jnp.exp(x), axis=1)
    mlp = (m - jnp.log(s)).reshape(BT, N)
    a = a_flat.reshape(BT, N)

    carry = jnp.where(step == 0, jnp.full((1, N), -1, jnp.int32),
                      prev_ref[0:1, :N])
    a_prev = jnp.concatenate([carry, a[:-1, :]], axis=0)
    prev_ref[0:1, :N] = a[-1:, :]

    tloc = lax.broadcasted_iota(jnp.int32, (BT, N), 0) + step * BT
    tmask = tloc < lens_ref[...]
    keep = (a != BLANK) & (a != a_prev) & tmask

    amax_ref[...] = (a | (keep.astype(jnp.int32) << 12)).T

    @pl.when(step == 0)
    def _():
        msum_ref[...] = jnp.zeros((1, N), jnp.float32)
        olen_ref[...] = jnp.zeros((1, N), jnp.int32)

    msum_ref[...] += jnp.sum(jnp.where(tmask, mlp, 0.0), axis=0)[None, :]
    olen_ref[...] += jnp.sum(keep.astype(jnp.int32), axis=0)[None, :]


def _stage1(logits, lens_row, interpret=False):
    return pl.pallas_call(
        _stage1_body,
        grid=(GRID,),
        in_specs=[
            pl.BlockSpec((1, N), lambda i: (0, 0)),
            pl.BlockSpec((BT * N, V), lambda i: (i, 0)),
        ],
        out_specs=[
            pl.BlockSpec((N, BT), lambda i: (0, i)),
            pl.BlockSpec((1, N), lambda i: (0, 0)),
            pl.BlockSpec((1, N), lambda i: (0, 0)),
        ],
        out_shape=[
            jax.ShapeDtypeStruct((N, T), jnp.int32),
            jax.ShapeDtypeStruct((1, N), jnp.float32),
            jax.ShapeDtypeStruct((1, N), jnp.int32),
        ],
        scratch_shapes=[pltpu.VMEM((8, 128), jnp.int32)],
        interpret=interpret,
    )(lens_row, logits.reshape(T * N, V))


def _stage2(packed_nt):
    mesh = plsc.VectorSubcoreMesh(core_axis_name="c", subcore_axis_name="s")

    @functools.partial(
        pl.kernel,
        out_type=jax.ShapeDtypeStruct((N, T), jnp.int32),
        mesh=mesh,
        scratch_types=[pltpu.VMEM((T,), jnp.int32)],
        compiler_params=pltpu.CompilerParams(needs_layout_passes=False),
    )
    def sc_kernel(packed_hbm, paths_hbm, a_v):
        cid = lax.axis_index("c")
        sid = lax.axis_index("s")
        row = cid * 8 + sid

        @pl.when(sid < 8)
        def _():
            pltpu.sync_copy(packed_hbm.at[row], a_v)

            def chunk(i, cnt):
                for u in range(2):
                    off = (2 * i + u) * LANES
                    p = a_v[pl.ds(off, LANES)]
                    k = p >> 12
                    a = p & 0xFFF
                    a_v[pl.ds(off, LANES)] = a
                    c = plsc.cumsum(k)
                    pos = cnt + c - 1
                    plsc.store_scatter(a_v, [pos], a, mask=k != 0)
                    cnt = cnt + c[LANES - 1]
                return cnt

            lax.fori_loop(0, T // LANES // 2, chunk, jnp.int32(0))
            pltpu.sync_copy(a_v, paths_hbm.at[row])

    return sc_kernel(packed_nt)


def kernel(logits, in_lens):
    lens_row = in_lens.reshape(1, N)
    packed_nt, msum, olen = _stage1(logits, lens_row)
    paths_nt = _stage2(packed_nt)
    return (msum.reshape(N), paths_nt.T, olen.reshape(N))


# speedup vs baseline: 1.0208x; 1.0001x over previous
import functools

import jax
import jax.numpy as jnp
from jax import lax
from jax.experimental import pallas as pl
from jax.experimental.pallas import tpu as pltpu
from jax.experimental.pallas import tpu_sc as plsc

T, N, V = 2048, 16, 1024
BLANK = V - 1
BT = 128
GRID = T // BT
LANES = 16


def _stage1_body(lens_ref, x_ref, amax_ref, msum_ref, olen_ref, prev_ref):
    step = pl.program_id(0)
    x = x_ref[...]
    m = jnp.max(x, axis=1)
    iota_v = lax.broadcasted_iota(jnp.int32, x.shape, 1).astype(jnp.float32)
    a_flat = jnp.min(jnp.where(x == m[:, None], iota_v, float(V)),
                     axis=1).astype(jnp.int32)
    s = jnp.sum(jnp.exp(x), axis=1)
    mlp = (m - jnp.log(s)).reshape(BT, N)
    a = a_flat.reshape(BT, N)

    carry = jnp.where(step == 0, jnp.full((1, N), -1, jnp.int32),
                      prev_ref[0:1, :N])
    a_prev = jnp.concatenate([carry, a[:-1, :]], axis=0)
    prev_ref[0:1, :N] = a[-1:, :]

    tloc = lax.broadcasted_iota(jnp.int32, (BT, N), 0) + step * BT
    tmask = tloc < lens_ref[...]
    keep = (a != BLANK) & (a != a_prev) & tmask

    amax_ref[...] = (a | (keep.astype(jnp.int32) << 12)).T

    @pl.when(step == 0)
    def _():
        msum_ref[...] = jnp.zeros((1, N), jnp.float32)
        olen_ref[...] = jnp.zeros((1, N), jnp.int32)

    msum_ref[...] += jnp.sum(jnp.where(tmask, mlp, 0.0), axis=0)[None, :]
    olen_ref[...] += jnp.sum(keep.astype(jnp.int32), axis=0)[None, :]


def _stage1(logits, lens_row):
    return pl.pallas_call(
        _stage1_body,
        grid=(GRID,),
        in_specs=[
            pl.BlockSpec((1, N), lambda i: (0, 0)),
            pl.BlockSpec((BT * N, V), lambda i: (i, 0)),
        ],
        out_specs=[
            pl.BlockSpec((N, BT), lambda i: (0, i)),
            pl.BlockSpec((1, N), lambda i: (0, 0)),
            pl.BlockSpec((1, N), lambda i: (0, 0)),
        ],
        out_shape=[
            jax.ShapeDtypeStruct((N, T), jnp.int32),
            jax.ShapeDtypeStruct((1, N), jnp.float32),
            jax.ShapeDtypeStruct((1, N), jnp.int32),
        ],
        scratch_shapes=[pltpu.VMEM((8, 128), jnp.int32)],
    )(lens_row, logits.reshape(T * N, V))


def _stage2(packed_nt):
    mesh = plsc.VectorSubcoreMesh(core_axis_name="c", subcore_axis_name="s")

    @functools.partial(
        pl.kernel,
        out_type=jax.ShapeDtypeStruct((N, T), jnp.int32),
        mesh=mesh,
        scratch_types=[pltpu.VMEM((T,), jnp.int32)],
        compiler_params=pltpu.CompilerParams(needs_layout_passes=False),
    )
    def sc_kernel(packed_hbm, paths_hbm, a_v):
        cid = lax.axis_index("c")
        sid = lax.axis_index("s")
        row = cid * 8 + sid

        @pl.when(sid < 8)
        def _():
            pltpu.sync_copy(packed_hbm.at[row], a_v)

            def chunk(i, cnt):
                for u in range(2):
                    off = (2 * i + u) * LANES
                    p = a_v[pl.ds(off, LANES)]
                    k = p >> 12
                    a = p & 0xFFF
                    a_v[pl.ds(off, LANES)] = a
                    c = plsc.cumsum(k)
                    pos = cnt + c - 1
                    plsc.store_scatter(a_v, [pos], a, mask=k != 0)
                    cnt = cnt + c[LANES - 1]
                return cnt

            lax.fori_loop(0, T // LANES // 2, chunk, jnp.int32(0))
            pltpu.sync_copy(a_v, paths_hbm.at[row])

    return sc_kernel(packed_nt)


def kernel(logits, in_lens):
    lens_row = in_lens.reshape(1, N)
    packed_nt, msum, olen = _stage1(logits, lens_row)
    paths_nt = _stage2(packed_nt)
    return (msum.reshape(N), paths_nt.T, olen.reshape(N))
